# baseline (speedup 1.0000x reference)
import functools

import jax
import jax.numpy as jnp
from jax import lax
from jax.experimental import pallas as pl
from jax.experimental.pallas import tpu as pltpu
from jax.experimental.pallas import tpu_sc as plsc

N_NODES = 10000
N_EDGES = 320000
D_FEAT = 128
N_HID = 64
N_CLASS = 41
C_PAD = 48
P1W = 80

NC = 2
NS = 16
NW = NC * NS
EW = N_EDGES // NW
CHUNK = 50
KG = 2
GROUP = KG * CHUNK
NG = EW // GROUP
ROWS_PER_W = EW // CHUNK
NPAD = 10240
RPW = NPAD // NS
WB = 624


def _zero_vmem_2d(ref, n_rows, n_cols):
  def row(r, _):
    for j in range(n_cols // 16):
      ref[r, pl.ds(j * 16, 16)] = jnp.zeros((16,), jnp.float32)
    return _
  lax.fori_loop(0, n_rows, row, 0)


def _make_sc_agg(feat: int):
  mesh = plsc.VectorSubcoreMesh(core_axis_name="c", subcore_axis_name="s")
  out_type = jax.ShapeDtypeStruct((NC, N_NODES, feat), jnp.float32)
  scratch = [
      [pltpu.VMEM((KG, CHUNK), jnp.int32) for _ in range(2)],
      [pltpu.VMEM((KG, CHUNK), jnp.int32) for _ in range(2)],
      [[pltpu.VMEM((CHUNK, feat), jnp.float32) for _ in range(KG)]
       for _ in range(2)],
      pltpu.VMEM_SHARED((NPAD, feat), jnp.float32),
      pltpu.VMEM_SHARED((N_NODES, feat), jnp.float32),
      pltpu.SemaphoreType.DMA,
      pltpu.SemaphoreType.DMA,
      pltpu.SemaphoreType.DMA,
  ]

  def body(p_hbm, edges_hbm, out_hbm, idx_s, idx_d, rows, acc, table,
           sem_i, sem_g, sem_s):
    src_hbm = edges_hbm.at[0]
    dst_hbm = edges_hbm.at[1]
    c = lax.axis_index("c")
    s = lax.axis_index("s")
    w = c * NS + s

    def idx_descs(slot, grow):
      return (pltpu.make_async_copy(src_hbm.at[pl.ds(grow, KG)], idx_s[slot],
                                    sem_i),
              pltpu.make_async_copy(dst_hbm.at[pl.ds(grow, KG)], idx_d[slot],
                                    sem_i))

    def fire_idx(slot, g):
      grow = w * ROWS_PER_W + g * KG
      for d in idx_descs(slot, grow):
        d.start()

    def drain_idx(slot):
      for d in idx_descs(slot, 0):
        d.wait()

    def gather_desc(rslot, qslot, j):
      return pltpu.make_async_copy(table.at[idx_s[qslot].at[j]],
                                   rows[rslot][j], sem_g)

    def drain_scatters(rslot, qslot):
      for j in range(KG):
        pltpu.make_async_copy(rows[rslot][j], acc.at[idx_d[qslot].at[j]],
                              sem_s).wait()

    st = s * WB
    pltpu.sync_copy(p_hbm.at[pl.ds(st, WB)], table.at[pl.ds(st, WB)])

    @pl.when(s == NS - 1)
    def _():
      pltpu.sync_copy(p_hbm.at[pl.ds(NS * WB, N_NODES - NS * WB)],
                      table.at[pl.ds(NS * WB, N_NODES - NS * WB)])

    zbuf = rows[0][0]
    _zero_vmem_2d(zbuf, CHUNK, feat)
    zsub = zbuf.at[pl.ds(0, 40)]
    zbase = s * RPW
    for k in range(RPW // 40):
      pltpu.sync_copy(zsub, acc.at[pl.ds(zbase + k * 40, 40)])
    plsc.subcore_barrier()

    fire_idx(0, 0)

    def pair_body(gi, carry):
      for phase in range(2):
        g = gi * 2 + phase
        slot = phase
        other = 1 - phase

        @pl.when(g < NG)
        def _():
          drain_idx(slot)
          for j in range(KG):
            gather_desc(slot, slot, j).start()

          @pl.when(g >= 1)
          def _():
            drain_scatters(other, other)

          @pl.when(g + 1 < NG)
          def _():
            fire_idx(other, g + 1)

          for j in range(KG):
            gather_desc(slot, slot, j).wait()
          for j in range(KG):
            pltpu.async_copy(rows[slot][j], acc.at[idx_d[slot].at[j]], sem_s,
                             add=True)
      return carry
    lax.fori_loop(0, (NG + 1) // 2, pair_body, 0)
    drain_scatters((NG - 1) % 2, (NG - 1) % 2)

    plsc.subcore_barrier()
    wb = s * WB
    pltpu.sync_copy(acc.at[pl.ds(wb, WB)], out_hbm.at[c, pl.ds(wb, WB)])

    @pl.when(s == NS - 1)
    def _():
      pltpu.sync_copy(acc.at[pl.ds(NS * WB, N_NODES - NS * WB)],
                      out_hbm.at[c, pl.ds(NS * WB, N_NODES - NS * WB)])

  return pl.kernel(body, out_type=out_type, mesh=mesh,
                   scratch_types=scratch,
                   compiler_params=pltpu.CompilerParams(
                       use_tc_tiling_on_sc=False))


_sc_agg_l1 = _make_sc_agg(P1W)
_sc_agg_l2 = _make_sc_agg(C_PAD)

_RB = 1000
_GRID = N_NODES // _RB


def _tc_p1_body(x_ref, wn_ref, p_ref):
  ones_col = lax.broadcasted_iota(jnp.int32, (_RB, P1W), 1) == N_HID
  p_ref[...] = (
      jnp.dot(x_ref[...], wn_ref[...], preferred_element_type=jnp.float32)
      + jnp.where(ones_col, 1.0, 0.0))


def _tc_p1(x, w1n_pad):
  return pl.pallas_call(
      _tc_p1_body,
      grid=(_GRID,),
      in_specs=[
          pl.BlockSpec((_RB, D_FEAT), lambda i: (i, 0)),
          pl.BlockSpec((D_FEAT, P1W), lambda i: (0, 0)),
      ],
      out_specs=pl.BlockSpec((_RB, P1W), lambda i: (i, 0)),
      out_shape=jax.ShapeDtypeStruct((N_NODES, P1W), jnp.float32),
  )(x, w1n_pad)


def _tc_s1_body(x_ref, ws_ref, s_ref):
  s_ref[...] = jnp.dot(x_ref[...], ws_ref[...],
                       preferred_element_type=jnp.float32)


def _tc_s1(x, w_self):
  return pl.pallas_call(
      _tc_s1_body,
      grid=(_GRID,),
      in_specs=[
          pl.BlockSpec((_RB, D_FEAT), lambda i: (i, 0)),
          pl.BlockSpec((D_FEAT, N_HID), lambda i: (0, 0)),
      ],
      out_specs=pl.BlockSpec((_RB, N_HID), lambda i: (i, 0)),
      out_shape=jax.ShapeDtypeStruct((N_NODES, N_HID), jnp.float32),
  )(x, w_self)


def _tc_mid_body(s1_ref, agg_ref, b1_ref, w2s_ref, w2n_ref, s2_ref, p2_ref):
  a = agg_ref[0] + agg_ref[1]
  deg = jnp.clip(a[:, N_HID:N_HID + 1], 1.0, None)
  mean = a[:, :N_HID] / deg
  h = jnp.maximum(s1_ref[...] + mean + b1_ref[...], 0.0)
  s2 = jnp.dot(h, w2s_ref[...], preferred_element_type=jnp.float32)
  deg_col = lax.broadcasted_iota(jnp.int32, (_RB, C_PAD), 1) == C_PAD - 1
  s2_ref[...] = jnp.where(deg_col, deg, s2)
  p2_ref[...] = jnp.dot(h, w2n_ref[...], preferred_element_type=jnp.float32)


def _tc_mid(s1, agg1, b1, w2s_pad, w2n_pad):
  return pl.pallas_call(
      _tc_mid_body,
      grid=(_GRID,),
      in_specs=[
          pl.BlockSpec((_RB, N_HID), lambda i: (i, 0)),
          pl.BlockSpec((NC, _RB, P1W), lambda i: (0, i, 0)),
          pl.BlockSpec((1, N_HID), lambda i: (0, 0)),
          pl.BlockSpec((N_HID, C_PAD), lambda i: (0, 0)),
          pl.BlockSpec((N_HID, C_PAD), lambda i: (0, 0)),
      ],
      out_specs=[
          pl.BlockSpec((_RB, C_PAD), lambda i: (i, 0)),
          pl.BlockSpec((_RB, C_PAD), lambda i: (i, 0)),
      ],
      out_shape=[
          jax.ShapeDtypeStruct((N_NODES, C_PAD), jnp.float32),
          jax.ShapeDtypeStruct((N_NODES, C_PAD), jnp.float32),
      ],
  )(s1, agg1, b1, w2s_pad, w2n_pad)


def _tc_out_body(s2_ref, agg_ref, b2_ref, out_ref):
  s2 = s2_ref[...]
  deg = s2[:, C_PAD - 1:C_PAD]
  z = s2 + (agg_ref[0] + agg_ref[1]) / deg + b2_ref[...]
  mask = lax.broadcasted_iota(jnp.int32, (_RB, C_PAD), 1) < N_CLASS
  zm = jnp.where(mask, z, -jnp.inf)
  m = jnp.max(zm, axis=-1, keepdims=True)
  e = jnp.where(mask, jnp.exp(zm - m), 0.0)
  lse = jnp.log(jnp.sum(e, axis=-1, keepdims=True)) + m
  out_ref[...] = (z - lse)[:, :N_CLASS]


def _tc_out(s2, agg2, b2_pad):
  return pl.pallas_call(
      _tc_out_body,
      grid=(_GRID,),
      in_specs=[
          pl.BlockSpec((_RB, C_PAD), lambda i: (i, 0)),
          pl.BlockSpec((NC, _RB, C_PAD), lambda i: (0, i, 0)),
          pl.BlockSpec((1, C_PAD), lambda i: (0, 0)),
      ],
      out_specs=pl.BlockSpec((_RB, N_CLASS), lambda i: (i, 0)),
      out_shape=jax.ShapeDtypeStruct((N_NODES, N_CLASS), jnp.float32),
  )(s2, agg2, b2_pad)


@jax.jit
def kernel(feature, edge_index, W1_self, W1_neigh, b1, W2_self, W2_neigh, b2):
  edges = edge_index.astype(jnp.int32).reshape(2, N_EDGES // CHUNK, CHUNK)

  w1n = jnp.pad(W1_neigh, ((0, 0), (0, P1W - N_HID)))
  p1 = _tc_p1(feature, w1n)
  agg1 = _sc_agg_l1(p1, edges)
  s1 = _tc_s1(feature, W1_self)

  w2s = jnp.pad(W2_self, ((0, 0), (0, C_PAD - N_CLASS)))
  w2n = jnp.pad(W2_neigh, ((0, 0), (0, C_PAD - N_CLASS)))
  s2, p2 = _tc_mid(s1, agg1, b1.reshape(1, N_HID), w2s, w2n)

  agg2 = _sc_agg_l2(p2, edges)

  b2p = jnp.pad(b2, (0, C_PAD - N_CLASS)).reshape(1, C_PAD)
  return _tc_out(s2, agg2, b2p)


# speedup vs baseline: 1.3589x; 1.3589x over previous
import functools

import jax
import jax.numpy as jnp
from jax import lax
from jax.experimental import pallas as pl
from jax.experimental.pallas import tpu as pltpu
from jax.experimental.pallas import tpu_sc as plsc

N_NODES = 10000
N_EDGES = 320000
D_FEAT = 128
N_HID = 64
N_CLASS = 41
C_PAD = 48
P1W = 80

NC = 2
NS = 16
NW = NC * NS
EW = N_EDGES // NW
CHUNK = 80
KG = 5
GROUP = KG * CHUNK
NG = EW // GROUP
ROWS_PER_W = EW // CHUNK
NPAD = 10240
RPW = NPAD // NS
WB = 624


def _zero_vmem_2d(ref, n_rows, n_cols):
  def row(r, _):
    for j in range(n_cols // 16):
      ref[r, pl.ds(j * 16, 16)] = jnp.zeros((16,), jnp.float32)
    return _
  lax.fori_loop(0, n_rows, row, 0)


def _make_sc_agg(feat: int):
  mesh = plsc.VectorSubcoreMesh(core_axis_name="c", subcore_axis_name="s")
  out_type = jax.ShapeDtypeStruct((NC, N_NODES, feat), jnp.float32)
  scratch = [
      [pltpu.VMEM((KG, CHUNK), jnp.int32) for _ in range(2)],
      [pltpu.VMEM((KG, CHUNK), jnp.int32) for _ in range(2)],
      [[pltpu.VMEM((CHUNK, feat), jnp.float32) for _ in range(KG)]
       for _ in range(2)],
      pltpu.VMEM_SHARED((NPAD, feat), jnp.float32),
      pltpu.SemaphoreType.DMA,
      pltpu.SemaphoreType.DMA,
      pltpu.SemaphoreType.DMA,
  ]

  def body(p_hbm, edges_hbm, out_hbm, idx_s, idx_d, rows, acc,
           sem_i, sem_g, sem_s):
    src_hbm = edges_hbm.at[0]
    dst_hbm = edges_hbm.at[1]
    c = lax.axis_index("c")
    s = lax.axis_index("s")
    w = c * NS + s

    def idx_descs(slot, grow):
      return (pltpu.make_async_copy(src_hbm.at[pl.ds(grow, KG)], idx_s[slot],
                                    sem_i),
              pltpu.make_async_copy(dst_hbm.at[pl.ds(grow, KG)], idx_d[slot],
                                    sem_i))

    def fire_idx(slot, g):
      grow = w * ROWS_PER_W + g * KG
      for d in idx_descs(slot, grow):
        d.start()

    def drain_idx(slot):
      for d in idx_descs(slot, 0):
        d.wait()

    def gather_desc(rslot, qslot, j):
      return pltpu.make_async_copy(p_hbm.at[idx_s[qslot].at[j]],
                                   rows[rslot][j], sem_g)

    def drain_scatters(rslot, qslot):
      for j in range(KG):
        pltpu.make_async_copy(rows[rslot][j], acc.at[idx_d[qslot].at[j]],
                              sem_s).wait()

    zbuf = rows[0][0]
    _zero_vmem_2d(zbuf, CHUNK, feat)
    zbase = s * RPW
    for k in range(RPW // CHUNK):
      pltpu.sync_copy(zbuf, acc.at[pl.ds(zbase + k * CHUNK, CHUNK)])
    plsc.subcore_barrier()

    fire_idx(0, 0)

    def pair_body(gi, carry):
      for phase in range(2):
        g = gi * 2 + phase
        slot = phase
        other = 1 - phase

        @pl.when(g < NG)
        def _():
          drain_idx(slot)
          for j in range(KG):
            gather_desc(slot, slot, j).start()

          @pl.when(g >= 1)
          def _():
            drain_scatters(other, other)

          @pl.when(g + 1 < NG)
          def _():
            fire_idx(other, g + 1)

          for j in range(KG):
            gather_desc(slot, slot, j).wait()
          for j in range(KG):
            pltpu.async_copy(rows[slot][j], acc.at[idx_d[slot].at[j]], sem_s,
                             add=True)
      return carry
    lax.fori_loop(0, (NG + 1) // 2, pair_body, 0)
    drain_scatters((NG - 1) % 2, (NG - 1) % 2)

    plsc.subcore_barrier()
    wb = s * WB
    pltpu.sync_copy(acc.at[pl.ds(wb, WB)], out_hbm.at[c, pl.ds(wb, WB)])

    @pl.when(s == NS - 1)
    def _():
      pltpu.sync_copy(acc.at[pl.ds(NS * WB, N_NODES - NS * WB)],
                      out_hbm.at[c, pl.ds(NS * WB, N_NODES - NS * WB)])

  return pl.kernel(body, out_type=out_type, mesh=mesh,
                   scratch_types=scratch,
                   compiler_params=pltpu.CompilerParams(
                       use_tc_tiling_on_sc=False))


_sc_agg_l1 = _make_sc_agg(P1W)
_sc_agg_l2 = _make_sc_agg(C_PAD)

_RB = 1000
_GRID = N_NODES // _RB


def _tc_p1_body(x_ref, wn_ref, p_ref):
  ones_col = lax.broadcasted_iota(jnp.int32, (_RB, P1W), 1) == N_HID
  p_ref[...] = (
      jnp.dot(x_ref[...], wn_ref[...], preferred_element_type=jnp.float32)
      + jnp.where(ones_col, 1.0, 0.0))


def _tc_p1(x, w1n_pad):
  return pl.pallas_call(
      _tc_p1_body,
      grid=(_GRID,),
      in_specs=[
          pl.BlockSpec((_RB, D_FEAT), lambda i: (i, 0)),
          pl.BlockSpec((D_FEAT, P1W), lambda i: (0, 0)),
      ],
      out_specs=pl.BlockSpec((_RB, P1W), lambda i: (i, 0)),
      out_shape=jax.ShapeDtypeStruct((N_NODES, P1W), jnp.float32),
  )(x, w1n_pad)


def _tc_s1_body(x_ref, ws_ref, s_ref):
  s_ref[...] = jnp.dot(x_ref[...], ws_ref[...],
                       preferred_element_type=jnp.float32)


def _tc_s1(x, w_self):
  return pl.pallas_call(
      _tc_s1_body,
      grid=(_GRID,),
      in_specs=[
          pl.BlockSpec((_RB, D_FEAT), lambda i: (i, 0)),
          pl.BlockSpec((D_FEAT, N_HID), lambda i: (0, 0)),
      ],
      out_specs=pl.BlockSpec((_RB, N_HID), lambda i: (i, 0)),
      out_shape=jax.ShapeDtypeStruct((N_NODES, N_HID), jnp.float32),
  )(x, w_self)


def _tc_mid_body(s1_ref, agg_ref, b1_ref, w2s_ref, w2n_ref, s2_ref, p2_ref):
  a = agg_ref[0] + agg_ref[1]
  deg = jnp.clip(a[:, N_HID:N_HID + 1], 1.0, None)
  mean = a[:, :N_HID] / deg
  h = jnp.maximum(s1_ref[...] + mean + b1_ref[...], 0.0)
  s2 = jnp.dot(h, w2s_ref[...], preferred_element_type=jnp.float32)
  deg_col = lax.broadcasted_iota(jnp.int32, (_RB, C_PAD), 1) == C_PAD - 1
  s2_ref[...] = jnp.where(deg_col, deg, s2)
  p2_ref[...] = jnp.dot(h, w2n_ref[...], preferred_element_type=jnp.float32)


def _tc_mid(s1, agg1, b1, w2s_pad, w2n_pad):
  return pl.pallas_call(
      _tc_mid_body,
      grid=(_GRID,),
      in_specs=[
          pl.BlockSpec((_RB, N_HID), lambda i: (i, 0)),
          pl.BlockSpec((NC, _RB, P1W), lambda i: (0, i, 0)),
          pl.BlockSpec((1, N_HID), lambda i: (0, 0)),
          pl.BlockSpec((N_HID, C_PAD), lambda i: (0, 0)),
          pl.BlockSpec((N_HID, C_PAD), lambda i: (0, 0)),
      ],
      out_specs=[
          pl.BlockSpec((_RB, C_PAD), lambda i: (i, 0)),
          pl.BlockSpec((_RB, C_PAD), lambda i: (i, 0)),
      ],
      out_shape=[
          jax.ShapeDtypeStruct((N_NODES, C_PAD), jnp.float32),
          jax.ShapeDtypeStruct((N_NODES, C_PAD), jnp.float32),
      ],
  )(s1, agg1, b1, w2s_pad, w2n_pad)


def _tc_out_body(s2_ref, agg_ref, b2_ref, out_ref):
  s2 = s2_ref[...]
  deg = s2[:, C_PAD - 1:C_PAD]
  z = s2 + (agg_ref[0] + agg_ref[1]) / deg + b2_ref[...]
  mask = lax.broadcasted_iota(jnp.int32, (_RB, C_PAD), 1) < N_CLASS
  zm = jnp.where(mask, z, -jnp.inf)
  m = jnp.max(zm, axis=-1, keepdims=True)
  e = jnp.where(mask, jnp.exp(zm - m), 0.0)
  lse = jnp.log(jnp.sum(e, axis=-1, keepdims=True)) + m
  out_ref[...] = (z - lse)[:, :N_CLASS]


def _tc_out(s2, agg2, b2_pad):
  return pl.pallas_call(
      _tc_out_body,
      grid=(_GRID,),
      in_specs=[
          pl.BlockSpec((_RB, C_PAD), lambda i: (i, 0)),
          pl.BlockSpec((NC, _RB, C_PAD), lambda i: (0, i, 0)),
          pl.BlockSpec((1, C_PAD), lambda i: (0, 0)),
      ],
      out_specs=pl.BlockSpec((_RB, N_CLASS), lambda i: (i, 0)),
      out_shape=jax.ShapeDtypeStruct((N_NODES, N_CLASS), jnp.float32),
  )(s2, agg2, b2_pad)


@jax.jit
def kernel(feature, edge_index, W1_self, W1_neigh, b1, W2_self, W2_neigh, b2):
  edges = edge_index.astype(jnp.int32).reshape(2, N_EDGES // CHUNK, CHUNK)

  w1n = jnp.pad(W1_neigh, ((0, 0), (0, P1W - N_HID)))
  p1 = _tc_p1(feature, w1n)
  agg1 = _sc_agg_l1(p1, edges)
  s1 = _tc_s1(feature, W1_self)

  w2s = jnp.pad(W2_self, ((0, 0), (0, C_PAD - N_CLASS)))
  w2n = jnp.pad(W2_neigh, ((0, 0), (0, C_PAD - N_CLASS)))
  s2, p2 = _tc_mid(s1, agg1, b1.reshape(1, N_HID), w2s, w2n)

  agg2 = _sc_agg_l2(p2, edges)

  b2p = jnp.pad(b2, (0, C_PAD - N_CLASS)).reshape(1, C_PAD)
  return _tc_out(s2, agg2, b2p)
